# Initial kernel scaffold; baseline (speedup 1.0000x reference)
import jax
import jax.numpy as jnp
from jax.experimental import pallas as pl


def kernel(hidden_states, router_w, w1, v1, w2, shared_gate, shared_up, shared_down):
    raise NotImplementedError("write your pallas kernel here")


# speedup vs baseline: 1.0713x; 1.0713x over previous
import functools

import jax
import jax.numpy as jnp
from jax import lax
from jax.experimental import pallas as pl
from jax.experimental.pallas import tpu as pltpu
from jax.experimental.pallas import tpu_sc as plsc

T = 2048
D = 2048
E = 8
F = 1024
K = 2
FS = 2048
BT = 256
GE = T * K // BT + E
PE = GE * BT
FK = 512


def _router_body(x_ref, rw_ref, a1_ref, a2_ref, w1_ref, w2_ref):
    x = x_ref[...]
    logits = lax.dot_general(x, rw_ref[...], (((1,), (1,)), ((), ())),
                             preferred_element_type=jnp.float32)
    m = jnp.max(logits, axis=1, keepdims=True)
    p = jnp.exp(logits - m)
    scores = p / jnp.sum(p, axis=1, keepdims=True)
    e_iota = lax.broadcasted_iota(jnp.int32, scores.shape, 1)
    m1 = jnp.max(scores, axis=1, keepdims=True)
    a1 = jnp.min(jnp.where(scores == m1, e_iota, E), axis=1, keepdims=True)
    s2 = jnp.where(e_iota == a1, -1.0, scores)
    m2 = jnp.max(s2, axis=1, keepdims=True)
    a2 = jnp.min(jnp.where(s2 == m2, e_iota, E), axis=1, keepdims=True)
    tot = m1 + m2
    a1_ref[...] = a1
    a2_ref[...] = a2
    w1_ref[...] = m1 / tot
    w2_ref[...] = m2 / tot


def _router(x, router_w):
    nt = T // BT
    outs = (
        jax.ShapeDtypeStruct((T, 1), jnp.int32),
        jax.ShapeDtypeStruct((T, 1), jnp.int32),
        jax.ShapeDtypeStruct((T, 1), jnp.float32),
        jax.ShapeDtypeStruct((T, 1), jnp.float32),
    )
    o_spec = pl.BlockSpec((BT, 1), lambda i: (i, 0))
    return pl.pallas_call(
        _router_body,
        grid=(nt,),
        in_specs=[pl.BlockSpec((BT, D), lambda i: (i, 0)),
                  pl.BlockSpec((E, D), lambda i: (0, 0))],
        out_specs=(o_spec, o_spec, o_spec, o_spec),
        out_shape=outs,
    )(x, router_w)


def _gmm_body(be_ref, xs_ref, w1_ref, v1_ref, w2_ref, ys_ref):
    fk = pl.program_id(1)
    x = xs_ref[...]
    t1 = lax.dot_general(x, w1_ref[0], (((1,), (1,)), ((), ())),
                         preferred_element_type=jnp.float32)
    t2 = lax.dot_general(x, v1_ref[0], (((1,), (1,)), ((), ())),
                         preferred_element_type=jnp.float32)
    h = t1 * lax.logistic(t1) * t2
    y = lax.dot_general(h, w2_ref[0], (((1,), (0,)), ((), ())),
                        preferred_element_type=jnp.float32)

    @pl.when(fk == 0)
    def _():
        ys_ref[...] = y

    @pl.when(fk != 0)
    def _():
        ys_ref[...] += y


def _gmm(xs, w1, v1, w2, block_expert):
    grid_spec = pltpu.PrefetchScalarGridSpec(
        num_scalar_prefetch=1,
        grid=(GE, F // FK),
        in_specs=[
            pl.BlockSpec((BT, D), lambda g, f, be: (g, 0)),
            pl.BlockSpec((1, FK, D), lambda g, f, be: (be[g], f, 0)),
            pl.BlockSpec((1, FK, D), lambda g, f, be: (be[g], f, 0)),
            pl.BlockSpec((1, FK, D), lambda g, f, be: (be[g], f, 0)),
        ],
        out_specs=pl.BlockSpec((BT, D), lambda g, f, be: (g, 0)),
    )
    return pl.pallas_call(
        _gmm_body,
        grid_spec=grid_spec,
        out_shape=jax.ShapeDtypeStruct((PE, D), jnp.float32),
    )(block_expert, xs, w1, v1, w2)


def _shared_body(x_ref, sg_ref, su_ref, sd_ref, ysh_ref):
    c = pl.program_id(1)
    fk = pl.program_id(2)
    x = x_ref[...]
    t1 = lax.dot_general(x, sg_ref[0], (((1,), (1,)), ((), ())),
                         preferred_element_type=jnp.float32)
    t2 = lax.dot_general(x, su_ref[0], (((1,), (1,)), ((), ())),
                         preferred_element_type=jnp.float32)
    h = t1 * lax.logistic(t1) * t2
    y = lax.dot_general(h, sd_ref[...], (((1,), (1,)), ((), ())),
                        preferred_element_type=jnp.float32)

    @pl.when((c == 0) & (fk == 0))
    def _():
        ysh_ref[...] = y

    @pl.when((c != 0) | (fk != 0))
    def _():
        ysh_ref[...] += y


def _shared(x, shared_gate, shared_up, shared_down):
    BTS = 512
    sg = shared_gate.reshape(FS // F, F, D)
    su = shared_up.reshape(FS // F, F, D)
    nt = T // BTS
    nc = FS // F
    nf = F // FK
    return pl.pallas_call(
        _shared_body,
        grid=(nt, nc, nf),
        in_specs=[
            pl.BlockSpec((BTS, D), lambda t, c, f: (t, 0)),
            pl.BlockSpec((1, FK, D), lambda t, c, f: (c, f, 0)),
            pl.BlockSpec((1, FK, D), lambda t, c, f: (c, f, 0)),
            pl.BlockSpec((D, FK), lambda t, c, f: (0, c * 2 + f)),
        ],
        out_specs=pl.BlockSpec((BTS, D), lambda t, c, f: (t, 0)),
        out_shape=jax.ShapeDtypeStruct((T, D), jnp.float32),
    )(x, sg, su, shared_down)


def kernel(hidden_states, router_w, w1, v1, w2, shared_gate, shared_up,
           shared_down):
    x = hidden_states.reshape(T, D)

    a1, a2, wt1, wt2 = _router(x, router_w)
    a1 = a1[:, 0]
    a2 = a2[:, 0]
    wt1 = wt1[:, 0]
    wt2 = wt2[:, 0]

    ee = jnp.stack([a1, a2], axis=1).reshape(-1)
    onehot = (ee[:, None] == jnp.arange(E)[None, :]).astype(jnp.int32)
    counts = jnp.sum(onehot, axis=0)
    rank = jnp.take_along_axis(jnp.cumsum(onehot, axis=0) - onehot,
                               ee[:, None], axis=1)[:, 0]
    nblk = (counts + BT - 1) // BT
    blk_start = jnp.concatenate([jnp.zeros((1,), jnp.int32),
                                 jnp.cumsum(nblk).astype(jnp.int32)])
    off = blk_start[:E] * BT
    pos = jnp.take(off, ee) + rank
    src_tok = jnp.zeros((PE,), jnp.int32).at[pos].set(
        jnp.arange(T * K, dtype=jnp.int32) // K)
    g_iota = jnp.arange(GE, dtype=jnp.int32)
    block_expert = jnp.clip(
        jnp.sum((g_iota[:, None] >= blk_start[None, :E]).astype(jnp.int32),
                axis=1) - 1, 0, E - 1)
    posk = pos.reshape(T, K)
    p1 = posk[:, 0]
    p2 = posk[:, 1]

    xs = jnp.take(x, src_tok, axis=0)

    ys = _gmm(xs, w1, v1, w2, block_expert)
    ysh = _shared(x, shared_gate, shared_up, shared_down)

    out = (wt1[:, None] * jnp.take(ys, p1, axis=0)
           + wt2[:, None] * jnp.take(ys, p2, axis=0) + ysh)
    return out.reshape(1, T, D)


# speedup vs baseline: 1.0737x; 1.0022x over previous
import functools

import jax
import jax.numpy as jnp
from jax import lax
from jax.experimental import pallas as pl
from jax.experimental.pallas import tpu as pltpu
from jax.experimental.pallas import tpu_sc as plsc

T = 2048
D = 2048
E = 8
F = 1024
K = 2
FS = 2048
BT = 256
GE = T * K // BT + E
PE = GE * BT
FK = 512


def _router_body(x_ref, rw_ref, a1_ref, a2_ref, w1_ref, w2_ref):
    x = x_ref[...]
    logits = lax.dot_general(x, rw_ref[...], (((1,), (1,)), ((), ())),
                             preferred_element_type=jnp.float32)
    m = jnp.max(logits, axis=1, keepdims=True)
    p = jnp.exp(logits - m)
    scores = p / jnp.sum(p, axis=1, keepdims=True)
    e_iota = lax.broadcasted_iota(jnp.int32, scores.shape, 1)
    m1 = jnp.max(scores, axis=1, keepdims=True)
    a1 = jnp.min(jnp.where(scores == m1, e_iota, E), axis=1, keepdims=True)
    s2 = jnp.where(e_iota == a1, -1.0, scores)
    m2 = jnp.max(s2, axis=1, keepdims=True)
    a2 = jnp.min(jnp.where(s2 == m2, e_iota, E), axis=1, keepdims=True)
    tot = m1 + m2
    a1_ref[...] = a1
    a2_ref[...] = a2
    w1_ref[...] = m1 / tot
    w2_ref[...] = m2 / tot


def _router(x, router_w):
    nt = T // BT
    outs = (
        jax.ShapeDtypeStruct((T, 1), jnp.int32),
        jax.ShapeDtypeStruct((T, 1), jnp.int32),
        jax.ShapeDtypeStruct((T, 1), jnp.float32),
        jax.ShapeDtypeStruct((T, 1), jnp.float32),
    )
    o_spec = pl.BlockSpec((BT, 1), lambda i: (i, 0))
    return pl.pallas_call(
        _router_body,
        grid=(nt,),
        in_specs=[pl.BlockSpec((BT, D), lambda i: (i, 0)),
                  pl.BlockSpec((E, D), lambda i: (0, 0))],
        out_specs=(o_spec, o_spec, o_spec, o_spec),
        out_shape=outs,
    )(x, router_w)


def _gmm_body(be_ref, xs_ref, w1_ref, v1_ref, w2_ref, ys_ref):
    fk = pl.program_id(1)
    x = xs_ref[...].astype(jnp.bfloat16)
    t1 = lax.dot_general(x, w1_ref[0].astype(jnp.bfloat16),
                         (((1,), (1,)), ((), ())),
                         preferred_element_type=jnp.float32)
    t2 = lax.dot_general(x, v1_ref[0].astype(jnp.bfloat16),
                         (((1,), (1,)), ((), ())),
                         preferred_element_type=jnp.float32)
    h = t1 * lax.logistic(t1) * t2
    y = lax.dot_general(h.astype(jnp.bfloat16), w2_ref[0].astype(jnp.bfloat16),
                        (((1,), (0,)), ((), ())),
                        preferred_element_type=jnp.float32)

    @pl.when(fk == 0)
    def _():
        ys_ref[...] = y

    @pl.when(fk != 0)
    def _():
        ys_ref[...] += y


def _gmm(xs, w1, v1, w2, block_expert):
    grid_spec = pltpu.PrefetchScalarGridSpec(
        num_scalar_prefetch=1,
        grid=(GE, F // FK),
        in_specs=[
            pl.BlockSpec((BT, D), lambda g, f, be: (g, 0)),
            pl.BlockSpec((1, FK, D), lambda g, f, be: (be[g], f, 0)),
            pl.BlockSpec((1, FK, D), lambda g, f, be: (be[g], f, 0)),
            pl.BlockSpec((1, FK, D), lambda g, f, be: (be[g], f, 0)),
        ],
        out_specs=pl.BlockSpec((BT, D), lambda g, f, be: (g, 0)),
    )
    return pl.pallas_call(
        _gmm_body,
        grid_spec=grid_spec,
        out_shape=jax.ShapeDtypeStruct((PE, D), jnp.float32),
    )(block_expert, xs, w1, v1, w2)


def _shared_body(x_ref, sg_ref, su_ref, sd_ref, ysh_ref):
    c = pl.program_id(1)
    fk = pl.program_id(2)
    x = x_ref[...].astype(jnp.bfloat16)
    t1 = lax.dot_general(x, sg_ref[0].astype(jnp.bfloat16),
                         (((1,), (1,)), ((), ())),
                         preferred_element_type=jnp.float32)
    t2 = lax.dot_general(x, su_ref[0].astype(jnp.bfloat16),
                         (((1,), (1,)), ((), ())),
                         preferred_element_type=jnp.float32)
    h = t1 * lax.logistic(t1) * t2
    y = lax.dot_general(h.astype(jnp.bfloat16), sd_ref[...].astype(jnp.bfloat16),
                        (((1,), (1,)), ((), ())),
                        preferred_element_type=jnp.float32)

    @pl.when((c == 0) & (fk == 0))
    def _():
        ysh_ref[...] = y

    @pl.when((c != 0) | (fk != 0))
    def _():
        ysh_ref[...] += y


def _shared(x, shared_gate, shared_up, shared_down):
    BTS = 512
    sg = shared_gate.reshape(FS // F, F, D)
    su = shared_up.reshape(FS // F, F, D)
    nt = T // BTS
    nc = FS // F
    nf = F // FK
    return pl.pallas_call(
        _shared_body,
        grid=(nt, nc, nf),
        in_specs=[
            pl.BlockSpec((BTS, D), lambda t, c, f: (t, 0)),
            pl.BlockSpec((1, FK, D), lambda t, c, f: (c, f, 0)),
            pl.BlockSpec((1, FK, D), lambda t, c, f: (c, f, 0)),
            pl.BlockSpec((D, FK), lambda t, c, f: (0, c * 2 + f)),
        ],
        out_specs=pl.BlockSpec((BTS, D), lambda t, c, f: (t, 0)),
        out_shape=jax.ShapeDtypeStruct((T, D), jnp.float32),
    )(x, sg, su, shared_down)


def kernel(hidden_states, router_w, w1, v1, w2, shared_gate, shared_up,
           shared_down):
    x = hidden_states.reshape(T, D)

    a1, a2, wt1, wt2 = _router(x, router_w)
    a1 = a1[:, 0]
    a2 = a2[:, 0]
    wt1 = wt1[:, 0]
    wt2 = wt2[:, 0]

    ee = jnp.stack([a1, a2], axis=1).reshape(-1)
    onehot = (ee[:, None] == jnp.arange(E)[None, :]).astype(jnp.int32)
    counts = jnp.sum(onehot, axis=0)
    rank = jnp.take_along_axis(jnp.cumsum(onehot, axis=0) - onehot,
                               ee[:, None], axis=1)[:, 0]
    nblk = (counts + BT - 1) // BT
    blk_start = jnp.concatenate([jnp.zeros((1,), jnp.int32),
                                 jnp.cumsum(nblk).astype(jnp.int32)])
    off = blk_start[:E] * BT
    pos = jnp.take(off, ee) + rank
    src_tok = jnp.zeros((PE,), jnp.int32).at[pos].set(
        jnp.arange(T * K, dtype=jnp.int32) // K)
    g_iota = jnp.arange(GE, dtype=jnp.int32)
    block_expert = jnp.clip(
        jnp.sum((g_iota[:, None] >= blk_start[None, :E]).astype(jnp.int32),
                axis=1) - 1, 0, E - 1)
    posk = pos.reshape(T, K)
    p1 = posk[:, 0]
    p2 = posk[:, 1]

    xs = jnp.take(x, src_tok, axis=0)

    ys = _gmm(xs, w1, v1, w2, block_expert)
    ysh = _shared(x, shared_gate, shared_up, shared_down)

    out = (wt1[:, None] * jnp.take(ys, p1, axis=0)
           + wt2[:, None] * jnp.take(ys, p2, axis=0) + ysh)
    return out.reshape(1, T, D)


# speedup vs baseline: 1.1293x; 1.0518x over previous
import functools

import jax
import jax.numpy as jnp
from jax import lax
from jax.experimental import pallas as pl
from jax.experimental.pallas import tpu as pltpu
from jax.experimental.pallas import tpu_sc as plsc

T = 2048
D = 2048
E = 8
F = 1024
K = 2
FS = 2048
BT = 128
GE = T * K // BT + E
PE = GE * BT
FK = 256


def _router_body(x_ref, rw_ref, a1_ref, a2_ref, w1_ref, w2_ref):
    x = x_ref[...]
    logits = lax.dot_general(x, rw_ref[...], (((1,), (1,)), ((), ())),
                             preferred_element_type=jnp.float32)
    m = jnp.max(logits, axis=1, keepdims=True)
    p = jnp.exp(logits - m)
    scores = p / jnp.sum(p, axis=1, keepdims=True)
    e_iota = lax.broadcasted_iota(jnp.int32, scores.shape, 1)
    m1 = jnp.max(scores, axis=1, keepdims=True)
    a1 = jnp.min(jnp.where(scores == m1, e_iota, E), axis=1, keepdims=True)
    s2 = jnp.where(e_iota == a1, -1.0, scores)
    m2 = jnp.max(s2, axis=1, keepdims=True)
    a2 = jnp.min(jnp.where(s2 == m2, e_iota, E), axis=1, keepdims=True)
    tot = m1 + m2
    a1_ref[...] = a1
    a2_ref[...] = a2
    w1_ref[...] = m1 / tot
    w2_ref[...] = m2 / tot


def _router(x, router_w):
    nt = T // BT
    outs = (
        jax.ShapeDtypeStruct((T, 1), jnp.int32),
        jax.ShapeDtypeStruct((T, 1), jnp.int32),
        jax.ShapeDtypeStruct((T, 1), jnp.float32),
        jax.ShapeDtypeStruct((T, 1), jnp.float32),
    )
    o_spec = pl.BlockSpec((BT, 1), lambda i: (i, 0))
    return pl.pallas_call(
        _router_body,
        grid=(nt,),
        in_specs=[pl.BlockSpec((BT, D), lambda i: (i, 0)),
                  pl.BlockSpec((E, D), lambda i: (0, 0))],
        out_specs=(o_spec, o_spec, o_spec, o_spec),
        out_shape=outs,
    )(x, router_w)


def _gmm_body(be_ref, xs_ref, w1_ref, v1_ref, w2_ref, ys_ref):
    x = xs_ref[...].astype(jnp.bfloat16)
    t1 = lax.dot_general(x, w1_ref[0].astype(jnp.bfloat16),
                         (((1,), (1,)), ((), ())),
                         preferred_element_type=jnp.float32)
    t2 = lax.dot_general(x, v1_ref[0].astype(jnp.bfloat16),
                         (((1,), (1,)), ((), ())),
                         preferred_element_type=jnp.float32)
    h = t1 * lax.logistic(t1) * t2
    ys_ref[...] = lax.dot_general(
        h.astype(jnp.bfloat16), w2_ref[0].astype(jnp.bfloat16),
        (((1,), (0,)), ((), ())), preferred_element_type=jnp.float32)


def _gmm(xs, w1, v1, w2, block_expert):
    grid_spec = pltpu.PrefetchScalarGridSpec(
        num_scalar_prefetch=1,
        grid=(GE,),
        in_specs=[
            pl.BlockSpec((BT, D), lambda g, be: (g, 0)),
            pl.BlockSpec((1, F, D), lambda g, be: (be[g], 0, 0)),
            pl.BlockSpec((1, F, D), lambda g, be: (be[g], 0, 0)),
            pl.BlockSpec((1, F, D), lambda g, be: (be[g], 0, 0)),
        ],
        out_specs=pl.BlockSpec((BT, D), lambda g, be: (g, 0)),
    )
    return pl.pallas_call(
        _gmm_body,
        grid_spec=grid_spec,
        out_shape=jax.ShapeDtypeStruct((PE, D), jnp.float32),
    )(block_expert, xs, w1, v1, w2)


def _shared_body(x_ref, sg_ref, su_ref, sd_ref, ysh_ref):
    c = pl.program_id(1)
    fk = pl.program_id(2)
    x = x_ref[...].astype(jnp.bfloat16)
    t1 = lax.dot_general(x, sg_ref[0].astype(jnp.bfloat16),
                         (((1,), (1,)), ((), ())),
                         preferred_element_type=jnp.float32)
    t2 = lax.dot_general(x, su_ref[0].astype(jnp.bfloat16),
                         (((1,), (1,)), ((), ())),
                         preferred_element_type=jnp.float32)
    h = t1 * lax.logistic(t1) * t2
    y = lax.dot_general(h.astype(jnp.bfloat16), sd_ref[...].astype(jnp.bfloat16),
                        (((1,), (1,)), ((), ())),
                        preferred_element_type=jnp.float32)

    @pl.when((c == 0) & (fk == 0))
    def _():
        ysh_ref[...] = y

    @pl.when((c != 0) | (fk != 0))
    def _():
        ysh_ref[...] += y


def _shared(x, shared_gate, shared_up, shared_down):
    BTS = 1024
    sg = shared_gate.reshape(FS // F, F, D)
    su = shared_up.reshape(FS // F, F, D)
    nt = T // BTS
    nc = FS // F
    nf = F // FK
    return pl.pallas_call(
        _shared_body,
        grid=(nt, nc, nf),
        in_specs=[
            pl.BlockSpec((BTS, D), lambda t, c, f: (t, 0)),
            pl.BlockSpec((1, FK, D), lambda t, c, f: (c, f, 0)),
            pl.BlockSpec((1, FK, D), lambda t, c, f: (c, f, 0)),
            pl.BlockSpec((D, FK), lambda t, c, f: (0, c * (F // FK) + f)),
        ],
        out_specs=pl.BlockSpec((BTS, D), lambda t, c, f: (t, 0)),
        out_shape=jax.ShapeDtypeStruct((T, D), jnp.float32),
    )(x, sg, su, shared_down)


def kernel(hidden_states, router_w, w1, v1, w2, shared_gate, shared_up,
           shared_down):
    x = hidden_states.reshape(T, D)

    a1, a2, wt1, wt2 = _router(x, router_w)
    a1 = a1[:, 0]
    a2 = a2[:, 0]
    wt1 = wt1[:, 0]
    wt2 = wt2[:, 0]

    ee = jnp.stack([a1, a2], axis=1).reshape(-1)
    onehot = (ee[:, None] == jnp.arange(E)[None, :]).astype(jnp.int32)
    counts = jnp.sum(onehot, axis=0)
    rank = jnp.take_along_axis(jnp.cumsum(onehot, axis=0) - onehot,
                               ee[:, None], axis=1)[:, 0]
    nblk = (counts + BT - 1) // BT
    blk_start = jnp.concatenate([jnp.zeros((1,), jnp.int32),
                                 jnp.cumsum(nblk).astype(jnp.int32)])
    off = blk_start[:E] * BT
    pos = jnp.take(off, ee) + rank
    src_tok = jnp.zeros((PE,), jnp.int32).at[pos].set(
        jnp.arange(T * K, dtype=jnp.int32) // K)
    g_iota = jnp.arange(GE, dtype=jnp.int32)
    block_expert = jnp.clip(
        jnp.sum((g_iota[:, None] >= blk_start[None, :E]).astype(jnp.int32),
                axis=1) - 1, 0, E - 1)
    posk = pos.reshape(T, K)
    p1 = posk[:, 0]
    p2 = posk[:, 1]

    xs = jnp.take(x, src_tok, axis=0)

    ys = _gmm(xs, w1, v1, w2, block_expert)
    ysh = _shared(x, shared_gate, shared_up, shared_down)

    out = (wt1[:, None] * jnp.take(ys, p1, axis=0)
           + wt2[:, None] * jnp.take(ys, p2, axis=0) + ysh)
    return out.reshape(1, T, D)


# speedup vs baseline: 1.2974x; 1.1489x over previous
import functools

import jax
import jax.numpy as jnp
from jax import lax
from jax.experimental import pallas as pl
from jax.experimental.pallas import tpu as pltpu
from jax.experimental.pallas import tpu_sc as plsc

T = 2048
D = 2048
E = 8
F = 1024
K = 2
FS = 2048
BT = 256
GE = T * K // BT + E
PE = GE * BT
FK = 256


def _router_body(x_ref, rw_ref, a1_ref, a2_ref, w1_ref, w2_ref):
    x = x_ref[...]
    logits = lax.dot_general(x, rw_ref[...], (((1,), (1,)), ((), ())),
                             preferred_element_type=jnp.float32)
    m = jnp.max(logits, axis=1, keepdims=True)
    p = jnp.exp(logits - m)
    scores = p / jnp.sum(p, axis=1, keepdims=True)
    e_iota = lax.broadcasted_iota(jnp.int32, scores.shape, 1)
    m1 = jnp.max(scores, axis=1, keepdims=True)
    a1 = jnp.min(jnp.where(scores == m1, e_iota, E), axis=1, keepdims=True)
    s2 = jnp.where(e_iota == a1, -1.0, scores)
    m2 = jnp.max(s2, axis=1, keepdims=True)
    a2 = jnp.min(jnp.where(s2 == m2, e_iota, E), axis=1, keepdims=True)
    tot = m1 + m2
    a1_ref[...] = a1
    a2_ref[...] = a2
    w1_ref[...] = m1 / tot
    w2_ref[...] = m2 / tot


def _router(x, router_w):
    nt = T // BT
    outs = (
        jax.ShapeDtypeStruct((T, 1), jnp.int32),
        jax.ShapeDtypeStruct((T, 1), jnp.int32),
        jax.ShapeDtypeStruct((T, 1), jnp.float32),
        jax.ShapeDtypeStruct((T, 1), jnp.float32),
    )
    o_spec = pl.BlockSpec((BT, 1), lambda i: (i, 0))
    return pl.pallas_call(
        _router_body,
        grid=(nt,),
        in_specs=[pl.BlockSpec((BT, D), lambda i: (i, 0)),
                  pl.BlockSpec((E, D), lambda i: (0, 0))],
        out_specs=(o_spec, o_spec, o_spec, o_spec),
        out_shape=outs,
    )(x, router_w)


def _gmm_body(meta_ref, xs_ref, w1_ref, v1_ref, w2_ref, ys_ref):
    g = pl.program_id(0)

    @pl.when(g < meta_ref[GE])
    def _():
        x = xs_ref[...].astype(jnp.bfloat16)
        t1 = lax.dot_general(x, w1_ref[0].astype(jnp.bfloat16),
                             (((1,), (1,)), ((), ())),
                             preferred_element_type=jnp.float32)
        t2 = lax.dot_general(x, v1_ref[0].astype(jnp.bfloat16),
                             (((1,), (1,)), ((), ())),
                             preferred_element_type=jnp.float32)
        h = t1 * lax.logistic(t1) * t2
        ys_ref[...] = lax.dot_general(
            h.astype(jnp.bfloat16), w2_ref[0].astype(jnp.bfloat16),
            (((1,), (0,)), ((), ())), preferred_element_type=jnp.float32)


def _gmm(xs, w1, v1, w2, meta):
    grid_spec = pltpu.PrefetchScalarGridSpec(
        num_scalar_prefetch=1,
        grid=(GE,),
        in_specs=[
            pl.BlockSpec((BT, D), lambda g, m: (g, 0)),
            pl.BlockSpec((1, F, D), lambda g, m: (m[g], 0, 0)),
            pl.BlockSpec((1, F, D), lambda g, m: (m[g], 0, 0)),
            pl.BlockSpec((1, F, D), lambda g, m: (m[g], 0, 0)),
        ],
        out_specs=pl.BlockSpec((BT, D), lambda g, m: (g, 0)),
    )
    return pl.pallas_call(
        _gmm_body,
        grid_spec=grid_spec,
        out_shape=jax.ShapeDtypeStruct((PE, D), jnp.float32),
    )(meta, xs, w1, v1, w2)


def _shared_body(x_ref, sg_ref, su_ref, sd_ref, ysh_ref):
    c = pl.program_id(1)
    fk = pl.program_id(2)
    x = x_ref[...].astype(jnp.bfloat16)
    t1 = lax.dot_general(x, sg_ref[0].astype(jnp.bfloat16),
                         (((1,), (1,)), ((), ())),
                         preferred_element_type=jnp.float32)
    t2 = lax.dot_general(x, su_ref[0].astype(jnp.bfloat16),
                         (((1,), (1,)), ((), ())),
                         preferred_element_type=jnp.float32)
    h = t1 * lax.logistic(t1) * t2
    y = lax.dot_general(h.astype(jnp.bfloat16), sd_ref[...].astype(jnp.bfloat16),
                        (((1,), (1,)), ((), ())),
                        preferred_element_type=jnp.float32)

    @pl.when((c == 0) & (fk == 0))
    def _():
        ysh_ref[...] = y

    @pl.when((c != 0) | (fk != 0))
    def _():
        ysh_ref[...] += y


def _shared(x, shared_gate, shared_up, shared_down):
    BTS = 1024
    sg = shared_gate.reshape(FS // F, F, D)
    su = shared_up.reshape(FS // F, F, D)
    nt = T // BTS
    nc = FS // F
    nf = F // FK
    return pl.pallas_call(
        _shared_body,
        grid=(nt, nc, nf),
        in_specs=[
            pl.BlockSpec((BTS, D), lambda t, c, f: (t, 0)),
            pl.BlockSpec((1, FK, D), lambda t, c, f: (c, f, 0)),
            pl.BlockSpec((1, FK, D), lambda t, c, f: (c, f, 0)),
            pl.BlockSpec((D, FK), lambda t, c, f: (0, c * (F // FK) + f)),
        ],
        out_specs=pl.BlockSpec((BTS, D), lambda t, c, f: (t, 0)),
        out_shape=jax.ShapeDtypeStruct((T, D), jnp.float32),
    )(x, sg, su, shared_down)


def kernel(hidden_states, router_w, w1, v1, w2, shared_gate, shared_up,
           shared_down):
    x = hidden_states.reshape(T, D)

    a1, a2, wt1, wt2 = _router(x, router_w)
    a1 = a1[:, 0]
    a2 = a2[:, 0]
    wt1 = wt1[:, 0]
    wt2 = wt2[:, 0]

    ee = jnp.stack([a1, a2], axis=1).reshape(-1)
    onehot = (ee[:, None] == jnp.arange(E)[None, :]).astype(jnp.int32)
    counts = jnp.sum(onehot, axis=0)
    rank = jnp.take_along_axis(jnp.cumsum(onehot, axis=0) - onehot,
                               ee[:, None], axis=1)[:, 0]
    nblk = (counts + BT - 1) // BT
    blk_start = jnp.concatenate([jnp.zeros((1,), jnp.int32),
                                 jnp.cumsum(nblk).astype(jnp.int32)])
    off = blk_start[:E] * BT
    pos = jnp.take(off, ee) + rank
    src_tok = jnp.zeros((PE,), jnp.int32).at[pos].set(
        jnp.arange(T * K, dtype=jnp.int32) // K)
    g_iota = jnp.arange(GE, dtype=jnp.int32)
    block_expert = jnp.clip(
        jnp.sum((g_iota[:, None] >= blk_start[None, :E]).astype(jnp.int32),
                axis=1) - 1, 0, E - 1)
    meta = jnp.concatenate([block_expert, blk_start[E:]])
    posk = pos.reshape(T, K)
    p1 = posk[:, 0]
    p2 = posk[:, 1]

    xs = jnp.take(x, src_tok, axis=0)

    ys = _gmm(xs, w1, v1, w2, meta)
    ysh = _shared(x, shared_gate, shared_up, shared_down)

    out = (wt1[:, None] * jnp.take(ys, p1, axis=0)
           + wt2[:, None] * jnp.take(ys, p2, axis=0) + ysh)
    return out.reshape(1, T, D)
